# initial kernel scaffold (unmeasured)
import jax
import jax.numpy as jnp
from jax import lax
from jax.experimental import pallas as pl
from jax.experimental.pallas import tpu as pltpu


def kernel(
    x,
):
    def body(*refs):
        pass

    out_shape = jax.ShapeDtypeStruct(..., jnp.float32)
    return pl.pallas_call(body, out_shape=out_shape)(...)


# baseline (device time: 198262 ns/iter reference)
import functools

import jax
import jax.numpy as jnp
from jax import lax
from jax.experimental import pallas as pl
from jax.experimental.pallas import tpu as pltpu


def kernel(x):
    m, n = x.shape

    def body(x_ref, out_ref, comm_ref, send_sem, recv_sem):
        my_x = lax.axis_index("x")
        my_y = lax.axis_index("y")
        my_z = lax.axis_index("z")
        peer = (my_x, 1 - my_y, my_z)

        barrier_sem = pltpu.get_barrier_semaphore()
        pl.semaphore_signal(
            barrier_sem, inc=1, device_id=peer,
            device_id_type=pltpu.DeviceIdType.MESH,
        )
        pl.semaphore_wait(barrier_sem, 1)

        rdma = pltpu.make_async_remote_copy(
            src_ref=x_ref,
            dst_ref=comm_ref,
            send_sem=send_sem,
            recv_sem=recv_sem,
            device_id=peer,
            device_id_type=pltpu.DeviceIdType.MESH,
        )
        rdma.start()
        rdma.wait()

        out_ref[:, :] = x_ref[:, :] + comm_ref[:, :]

        @functools.partial(
            pl.run_scoped, second_barrier=pltpu.SemaphoreType.REGULAR
        )
        def _(second_barrier):
            pl.semaphore_signal(
                second_barrier, inc=1, device_id=peer,
                device_id_type=pltpu.DeviceIdType.MESH,
            )
            pl.semaphore_wait(second_barrier, 1)

    return pl.pallas_call(
        body,
        out_shape=jax.ShapeDtypeStruct((m, n), x.dtype),
        in_specs=[pl.BlockSpec(memory_space=pltpu.VMEM)],
        out_specs=pl.BlockSpec(memory_space=pltpu.VMEM),
        scratch_shapes=[
            pltpu.VMEM((m, n), x.dtype),
            pltpu.SemaphoreType.DMA,
            pltpu.SemaphoreType.DMA,
        ],
        compiler_params=pltpu.CompilerParams(collective_id=0),
    )(x)


# device time: 115150 ns/iter; 1.7218x vs baseline; 1.7218x over previous
import functools

import jax
import jax.numpy as jnp
from jax import lax
from jax.experimental import pallas as pl
from jax.experimental.pallas import tpu as pltpu

N_CHUNKS = 16


def kernel(x):
    m, n = x.shape
    half = m // 2
    rows = half // N_CHUNKS

    def body(x_ref, out_ref, comm_ref,
             y_send_sems, y_recv_sems, z_send_sems, z_recv_sems):
        my_x = lax.axis_index("x")
        my_y = lax.axis_index("y")
        my_z = lax.axis_index("z")
        peer_y = (my_x, 1 - my_y, my_z)
        peer_z = (my_x, my_y, 1 - my_z)

        my_off = my_z * half
        other_off = half - my_off

        barrier_sem = pltpu.get_barrier_semaphore()
        for peer in (peer_y, peer_z):
            pl.semaphore_signal(
                barrier_sem, inc=1, device_id=peer,
                device_id_type=pltpu.DeviceIdType.MESH,
            )
        pl.semaphore_wait(barrier_sem, 2)

        y_rdmas = []
        for c in range(N_CHUNKS):
            sl = pl.ds(my_off + c * rows, rows)
            rdma = pltpu.make_async_remote_copy(
                src_ref=x_ref.at[sl, :],
                dst_ref=comm_ref.at[sl, :],
                send_sem=y_send_sems.at[c],
                recv_sem=y_recv_sems.at[c],
                device_id=peer_y,
                device_id_type=pltpu.DeviceIdType.MESH,
            )
            rdma.start()
            y_rdmas.append(rdma)

        z_rdmas = []
        for c in range(N_CHUNKS):
            sl = pl.ds(my_off + c * rows, rows)
            y_rdmas[c].wait_recv()
            fwd = pltpu.make_async_remote_copy(
                src_ref=comm_ref.at[sl, :],
                dst_ref=comm_ref.at[sl, :],
                send_sem=z_send_sems.at[c],
                recv_sem=z_recv_sems.at[c],
                device_id=peer_z,
                device_id_type=pltpu.DeviceIdType.MESH,
            )
            fwd.start()
            z_rdmas.append(fwd)
            out_ref[sl, :] = x_ref[sl, :] + comm_ref[sl, :]

        for c in range(N_CHUNKS):
            sl = pl.ds(other_off + c * rows, rows)
            recv = pltpu.make_async_remote_copy(
                src_ref=comm_ref.at[sl, :],
                dst_ref=comm_ref.at[sl, :],
                send_sem=z_send_sems.at[c],
                recv_sem=z_recv_sems.at[c],
                device_id=peer_z,
                device_id_type=pltpu.DeviceIdType.MESH,
            )
            recv.wait_recv()
            out_ref[sl, :] = x_ref[sl, :] + comm_ref[sl, :]

        for c in range(N_CHUNKS):
            y_rdmas[c].wait_send()
            z_rdmas[c].wait_send()

        @functools.partial(
            pl.run_scoped, second_barrier=pltpu.SemaphoreType.REGULAR
        )
        def _(second_barrier):
            for peer in (peer_y, peer_z):
                pl.semaphore_signal(
                    second_barrier, inc=1, device_id=peer,
                    device_id_type=pltpu.DeviceIdType.MESH,
                )
            pl.semaphore_wait(second_barrier, 2)

    return pl.pallas_call(
        body,
        out_shape=jax.ShapeDtypeStruct((m, n), x.dtype),
        in_specs=[pl.BlockSpec(memory_space=pltpu.VMEM)],
        out_specs=pl.BlockSpec(memory_space=pltpu.VMEM),
        scratch_shapes=[
            pltpu.VMEM((m, n), x.dtype),
            pltpu.SemaphoreType.DMA((N_CHUNKS,)),
            pltpu.SemaphoreType.DMA((N_CHUNKS,)),
            pltpu.SemaphoreType.DMA((N_CHUNKS,)),
            pltpu.SemaphoreType.DMA((N_CHUNKS,)),
        ],
        compiler_params=pltpu.CompilerParams(collective_id=0),
    )(x)


# device time: 83843 ns/iter; 2.3647x vs baseline; 1.3734x over previous
import functools

import jax
import jax.numpy as jnp
from jax import lax
from jax.experimental import pallas as pl
from jax.experimental.pallas import tpu as pltpu

QROWS = 1024

S0, S1, S2 = 448, 288, 288
PIECE_OFF = (0, S0, S0 + S1)
CHUNK_ROWS = ((112, 112, 112, 112), (96, 96, 96), (96, 96, 96))


def _chunks(pieces):
    out = []
    for q_rel, j in pieces:
        off = PIECE_OFF[j]
        for r in CHUNK_ROWS[j]:
            out.append((q_rel, off, r))
            off += r
    return out


Y_STREAM = _chunks([(0, 0), (0, 1), (0, 2), (2, 0)])
CW_OUT = _chunks([(0, 0), (0, 1), (0, 2), (3, 2)])
CCW_OUT = _chunks([(0, 0), (0, 1), (0, 2), (1, 1)])
CW_IN = _chunks([(3, 0), (3, 1), (3, 2), (2, 2)])
CCW_IN = _chunks([(1, 0), (1, 1), (1, 2), (2, 1)])

MESH = pltpu.DeviceIdType.MESH


def kernel(x):
    m, n = x.shape

    def body(x_ref, out_ref, comm_ref,
             y_s, y_r, cw_s, cw_r, ccw_s, ccw_r):
        my_x = lax.axis_index("x")
        my_y = lax.axis_index("y")
        my_z = lax.axis_index("z")
        k = 2 * my_x + (my_x ^ my_z)
        even = (k % 2) == 0
        nxt = (jnp.where(even, my_x, 1 - my_x), my_y,
               jnp.where(even, 1 - my_z, my_z))
        prv = (jnp.where(even, 1 - my_x, my_x), my_y,
               jnp.where(even, my_z, 1 - my_z))
        peer_y = (my_x, 1 - my_y, my_z)

        def rows(q_rel, off, nrows):
            return pl.ds(((k + q_rel) % 4) * QROWS + off, nrows)

        barrier_sem = pltpu.get_barrier_semaphore()
        for p in (peer_y, nxt, prv):
            pl.semaphore_signal(barrier_sem, inc=1, device_id=p,
                                device_id_type=MESH)
        pl.semaphore_wait(barrier_sem, 3)

        y_rd = []
        for c, (q_rel, off, nr) in enumerate(Y_STREAM):
            sl = rows(q_rel, off, nr)
            r = pltpu.make_async_remote_copy(
                src_ref=x_ref.at[sl, :], dst_ref=comm_ref.at[sl, :],
                send_sem=y_s.at[c], recv_sem=y_r.at[c],
                device_id=peer_y, device_id_type=MESH)
            r.start()
            y_rd.append(r)

        cw_rd = [None] * len(CW_OUT)
        ccw_rd = [None] * len(CCW_OUT)

        def ring_start(stream, sems_s, sems_r, target, c, lst):
            q_rel, off, nr = stream[c]
            sl = rows(q_rel, off, nr)
            r = pltpu.make_async_remote_copy(
                src_ref=comm_ref.at[sl, :], dst_ref=comm_ref.at[sl, :],
                send_sem=sems_s.at[c], recv_sem=sems_r.at[c],
                device_id=target, device_id_type=MESH)
            r.start()
            lst[c] = r

        def in_wait(stream, sems_r, c):
            q_rel, off, nr = stream[c]
            sl = rows(q_rel, off, nr)
            r = pltpu.make_async_remote_copy(
                src_ref=comm_ref.at[sl, :], dst_ref=comm_ref.at[sl, :],
                send_sem=y_s.at[0], recv_sem=sems_r.at[c],
                device_id=peer_y, device_id_type=MESH)
            r.wait_recv()

        def add(stream, c):
            q_rel, off, nr = stream[c]
            sl = rows(q_rel, off, nr)
            out_ref[sl, :] = x_ref[sl, :] + comm_ref[sl, :]

        def fold_y(c):
            y_rd[c].wait_recv()
            add(Y_STREAM, c)

        def fold_cw(c):
            in_wait(CW_IN, cw_r, c)
            if 7 <= c <= 9:
                ring_start(CW_OUT, cw_s, cw_r, nxt, c + 3, cw_rd)
            add(CW_IN, c)

        def fold_ccw(c):
            in_wait(CCW_IN, ccw_r, c)
            if 4 <= c <= 6:
                ring_start(CCW_OUT, ccw_s, ccw_r, prv, c + 6, ccw_rd)
            add(CCW_IN, c)

        for c in range(10):
            y_rd[c].wait_recv()
            ring_start(CW_OUT, cw_s, cw_r, nxt, c, cw_rd)
            ring_start(CCW_OUT, ccw_s, ccw_r, prv, c, ccw_rd)
            add(Y_STREAM, c)
            if c == 6:
                fold_ccw(4)
            elif c == 7:
                fold_ccw(5)
            elif c == 8:
                fold_ccw(6)
                fold_cw(7)
            elif c == 9:
                fold_cw(8)
                fold_cw(9)
        for c in range(4):
            fold_cw(c)
            fold_ccw(c)
        for c in range(4, 7):
            fold_cw(c)
        for c in range(7, 10):
            fold_ccw(c)
        for c in range(10, 14):
            fold_y(c)
        for c in range(10, 13):
            fold_cw(c)
            fold_ccw(c)

        for r in y_rd:
            r.wait_send()
        for r in cw_rd:
            r.wait_send()
        for r in ccw_rd:
            r.wait_send()

        @functools.partial(
            pl.run_scoped, second_barrier=pltpu.SemaphoreType.REGULAR
        )
        def _(second_barrier):
            for p in (peer_y, nxt, prv):
                pl.semaphore_signal(second_barrier, inc=1, device_id=p,
                                    device_id_type=MESH)
            pl.semaphore_wait(second_barrier, 3)

    nc = len(Y_STREAM)
    nr_ = len(CW_OUT)
    return pl.pallas_call(
        body,
        out_shape=jax.ShapeDtypeStruct((m, n), x.dtype),
        in_specs=[pl.BlockSpec(memory_space=pltpu.VMEM)],
        out_specs=pl.BlockSpec(memory_space=pltpu.VMEM),
        scratch_shapes=[
            pltpu.VMEM((m, n), x.dtype),
            pltpu.SemaphoreType.DMA((nc,)),
            pltpu.SemaphoreType.DMA((nc,)),
            pltpu.SemaphoreType.DMA((nr_,)),
            pltpu.SemaphoreType.DMA((nr_,)),
            pltpu.SemaphoreType.DMA((nr_,)),
            pltpu.SemaphoreType.DMA((nr_,)),
        ],
        compiler_params=pltpu.CompilerParams(collective_id=0),
    )(x)
